# baseline (speedup 1.0000x reference)
import functools

import jax
import jax.numpy as jnp
from jax import lax
from jax.experimental import pallas as pl
from jax.experimental.pallas import tpu as pltpu
from jax.experimental.pallas import tpu_sc as plsc

KNN = 40
NEG = -3.0e38


def _knn_body(x_ref, wp_ref, wq_ref, b_ref, idx_ref, p_ref, q_ref, d_ref,
              c_ref):
    bidx = pl.program_id(0)
    r = pl.program_id(1)
    R = idx_ref.shape[0]
    N = x_ref.shape[2]
    xb = x_ref[0]
    xi = x_ref[0, :, pl.ds(r * R, R)]
    xx = jnp.sum(xb * xb, axis=0, keepdims=True)
    ones = jnp.ones((3, 1), jnp.float32)
    xxi = lax.dot_general(xi * xi, ones, (((0,), (0,)), ((), ())))
    cross = lax.dot_general(xi, xb, (((0,), (0,)), ((), ())))
    p_ref[...] = lax.dot_general(xi, wp_ref[...], (((0,), (0,)), ((), ())))
    q_ref[...] = lax.dot_general(xi, wq_ref[...], (((0,), (0,)), ((), ()))) \
        + b_ref[...]
    iota = lax.broadcasted_iota(jnp.int32, (R, N), 1)
    iota128 = lax.broadcasted_iota(jnp.int32, (R, 128), 1)
    base = bidx * N
    d_raw = 2.0 * cross - xxi - xx
    db = lax.bitcast_convert_type(d_raw, jnp.int32)
    pk = lax.bitcast_convert_type((db & -16) | (iota >> 7), jnp.float32)
    self_idx = r * R + lax.broadcasted_iota(jnp.int32, (R, 1), 0)
    d0 = jnp.where(iota == self_idx, NEG, pk)
    idx_ref[:, 0:1] = self_idx + base
    d_ref[...] = d0

    def colmax(a):
        c = a[:, 0:128]
        for v in range(1, 16):
            c = jnp.maximum(c, a[:, v * 128:(v + 1) * 128])
        return c

    c_ref[...] = colmax(d0)

    def peak(cm):
        vmax = jnp.max(cm, axis=1, keepdims=True)
        vstar = lax.bitcast_convert_type(vmax, jnp.int32) & 15
        lstar = jnp.min(jnp.where(cm == vmax, iota128, 128),
                        axis=1, keepdims=True)
        return vmax, vstar * 128 + lstar, lstar

    for t in range(1, KNN - 1, 2):
        cmax = c_ref[...]
        vmax1, am1, l1 = peak(cmax)
        idx_ref[:, t:t + 1] = am1 + base
        cmask = jnp.where(iota128 == l1, NEG, cmax)
        vmax2, am2, _ = peak(cmask)
        idx_ref[:, t + 1:t + 2] = am2 + base
        cur = d_ref[...]
        sel = (cur == vmax1) | (cur == vmax2)
        newd = jnp.where(sel, NEG, cur)
        d_ref[...] = newd
        c_ref[...] = colmax(newd)
    cmax = c_ref[...]
    _, am1, _ = peak(cmax)
    idx_ref[:, KNN - 1:KNN] = am1 + base


def _knn_topk(x, wp, wq, bb):
    B, _, N = x.shape
    R = 256
    NB = N // R
    out_shape = [
        jax.ShapeDtypeStruct((B * N, KNN), jnp.int32),
        jax.ShapeDtypeStruct((B * N, 128), jnp.float32),
        jax.ShapeDtypeStruct((B * N, 64), jnp.float32),
    ]
    return pl.pallas_call(
        _knn_body,
        grid=(B, NB),
        in_specs=[
            pl.BlockSpec((1, 3, N), lambda b, r: (b, 0, 0)),
            pl.BlockSpec((3, 128), lambda b, r: (0, 0)),
            pl.BlockSpec((3, 64), lambda b, r: (0, 0)),
            pl.BlockSpec((1, 64), lambda b, r: (0, 0)),
        ],
        out_specs=[
            pl.BlockSpec((R, KNN), lambda b, r: (b * NB + r, 0)),
            pl.BlockSpec((R, 128), lambda b, r: (b * NB + r, 0)),
            pl.BlockSpec((R, 64), lambda b, r: (b * NB + r, 0)),
        ],
        out_shape=out_shape,
        scratch_shapes=[pltpu.VMEM((R, N), jnp.float32),
                        pltpu.VMEM((R, 128), jnp.float32)],
    )(x, wp, wq, bb)


CROWS = 8
NCH = None


def _sc_gather_max(idx, p, q):
    BN = idx.shape[0]
    info = plsc.get_sparse_core_info()
    nc, ns = info.num_cores, info.num_subcores
    nw = nc * ns
    rows_per_w = BN // nw
    nch = rows_per_w // CROWS
    mesh = plsc.VectorSubcoreMesh(core_axis_name="c", subcore_axis_name="s")

    @functools.partial(
        pl.kernel,
        mesh=mesh,
        out_type=jax.ShapeDtypeStruct((BN, 64), jnp.float32),
        scratch_types=[
            pltpu.VMEM((2, CROWS, KNN), jnp.int32),
            pltpu.VMEM((2, CROWS * KNN, 128), jnp.float32),
            pltpu.VMEM((2, CROWS, 64), jnp.float32),
            pltpu.VMEM((CROWS, 64), jnp.float32),
            pltpu.SemaphoreType.DMA,
            pltpu.SemaphoreType.DMA,
            pltpu.SemaphoreType.DMA,
            pltpu.SemaphoreType.DMA,
        ],
    )
    def body(idx_hbm, p_hbm, q_hbm, out_hbm, idx_v, rows_v, q_v, o_v,
             sg0, sg1, sq0, sq1):
        wid = lax.axis_index("s") * nc + lax.axis_index("c")
        row0 = wid * rows_per_w
        semg = (sg0, sg1)
        semq = (sq0, sq1)

        def stage(c, s):
            base = row0 + c * CROWS
            pltpu.sync_copy(idx_hbm.at[pl.ds(base, CROWS), :], idx_v.at[s])
            for j in range(CROWS):
                pltpu.async_copy(
                    p_hbm.at[idx_v.at[s, j]],
                    rows_v.at[s, pl.ds(j * KNN, KNN), :],
                    semg[s])
            pltpu.async_copy(q_hbm.at[pl.ds(base, CROWS), :], q_v.at[s],
                             semq[s])

        def drain(s):
            pltpu.make_async_copy(
                p_hbm.at[pl.ds(0, CROWS * KNN), :], rows_v.at[s],
                semg[s]).wait()
            pltpu.make_async_copy(
                q_hbm.at[pl.ds(0, CROWS), :], q_v.at[s], semq[s]).wait()

        def compute(c, s):
            base = row0 + c * CROWS
            for rr in range(CROWS):
                def jb(u, ms):
                    j0 = rr * KNN + u * 5
                    for uu in range(5):
                        ms = tuple(
                            jnp.maximum(ms[g],
                                        rows_v[s, j0 + uu, pl.ds(g * 16, 16)])
                            for g in range(4))
                    return ms
                init = tuple(jnp.full((16,), NEG, jnp.float32)
                             for _ in range(4))
                ms = lax.fori_loop(0, KNN // 5, jb, init)
                for g in range(4):
                    h = ms[g] + q_v[s, rr, pl.ds(g * 16, 16)]
                    o_v[rr, pl.ds(g * 16, 16)] = jnp.maximum(h, 0.2 * h)
            pltpu.sync_copy(o_v, out_hbm.at[pl.ds(base, CROWS), :])

        stage(0, 0)
        stage(1, 1)

        def outer(i, carry):
            c0 = 2 * i
            drain(0)
            compute(c0, 0)
            stage(c0 + 2, 0)
            drain(1)
            compute(c0 + 1, 1)
            stage(c0 + 3, 1)
            return carry

        lax.fori_loop(0, nch // 2 - 1, outer, 0)
        drain(0)
        compute(nch - 2, 0)
        drain(1)
        compute(nch - 1, 1)

    return body(idx, p, q)


def kernel(x, W, b):
    B, _, N = x.shape
    wp = jnp.zeros((3, 128), jnp.float32).at[:, :64].set(W[:, :3].T)
    wq = (W[:, 3:] - W[:, :3]).T
    bb = b.reshape(1, 64)
    idx, p, q = _knn_topk(x, wp, wq, bb)
    out_t = _sc_gather_max(idx, p, q)
    return out_t.reshape(B, N, 64).transpose(0, 2, 1)


# speedup vs baseline: 1.0976x; 1.0976x over previous
import functools

import jax
import jax.numpy as jnp
from jax import lax
from jax.experimental import pallas as pl
from jax.experimental.pallas import tpu as pltpu
from jax.experimental.pallas import tpu_sc as plsc

KNN = 40
NEG = -3.0e38


def _knn_body(x_ref, wp_ref, wq_ref, b_ref, idx_ref, p_ref, q_ref, d_ref,
              c_ref):
    bidx = pl.program_id(0)
    r = pl.program_id(1)
    R = idx_ref.shape[0]
    N = x_ref.shape[2]
    xb = x_ref[0]
    xi = x_ref[0, :, pl.ds(r * R, R)]
    xx = jnp.sum(xb * xb, axis=0, keepdims=True)
    ones = jnp.ones((3, 1), jnp.float32)
    xxi = lax.dot_general(xi * xi, ones, (((0,), (0,)), ((), ())))
    cross = lax.dot_general(xi, xb, (((0,), (0,)), ((), ())))
    p_ref[...] = lax.dot_general(xi, wp_ref[...], (((0,), (0,)), ((), ())))
    q_ref[...] = lax.dot_general(xi, wq_ref[...], (((0,), (0,)), ((), ()))) \
        + b_ref[...]
    iota = lax.broadcasted_iota(jnp.int32, (R, N), 1)
    iota128 = lax.broadcasted_iota(jnp.int32, (R, 128), 1)
    base = bidx * N
    d_raw = 2.0 * cross - xxi - xx
    db = lax.bitcast_convert_type(d_raw, jnp.int32)
    pk = lax.bitcast_convert_type((db & -16) | (iota >> 7), jnp.float32)
    self_idx = r * R + lax.broadcasted_iota(jnp.int32, (R, 1), 0)
    d0 = jnp.where(iota == self_idx, NEG, pk)
    idx_ref[:, 0:1] = self_idx + base
    d_ref[...] = d0

    def colmax(a):
        c = a[:, 0:128]
        for v in range(1, 16):
            c = jnp.maximum(c, a[:, v * 128:(v + 1) * 128])
        return c

    c_ref[...] = colmax(d0)

    def peak(cm):
        vmax = jnp.max(cm, axis=1, keepdims=True)
        vstar = lax.bitcast_convert_type(vmax, jnp.int32) & 15
        lstar = jnp.min(jnp.where(cm == vmax, iota128, 128),
                        axis=1, keepdims=True)
        return vmax, vstar * 128 + lstar, lstar

    for t in range(1, KNN - 1, 2):
        cmax = c_ref[...]
        vmax1, am1, l1 = peak(cmax)
        idx_ref[:, t:t + 1] = am1 + base
        cmask = jnp.where(iota128 == l1, NEG, cmax)
        vmax2, am2, _ = peak(cmask)
        idx_ref[:, t + 1:t + 2] = am2 + base
        cur = d_ref[...]
        sel = (cur == vmax1) | (cur == vmax2)
        newd = jnp.where(sel, NEG, cur)
        d_ref[...] = newd
        c_ref[...] = colmax(newd)
    cmax = c_ref[...]
    _, am1, _ = peak(cmax)
    idx_ref[:, KNN - 1:KNN] = am1 + base


def _knn_topk(x, wp, wq, bb):
    B, _, N = x.shape
    R = 256
    NB = N // R
    out_shape = [
        jax.ShapeDtypeStruct((B * N, KNN), jnp.int32),
        jax.ShapeDtypeStruct((B * N, 128), jnp.float32),
        jax.ShapeDtypeStruct((B * N, 64), jnp.float32),
    ]
    return pl.pallas_call(
        _knn_body,
        grid=(B, NB),
        in_specs=[
            pl.BlockSpec((1, 3, N), lambda b, r: (b, 0, 0)),
            pl.BlockSpec((3, 128), lambda b, r: (0, 0)),
            pl.BlockSpec((3, 64), lambda b, r: (0, 0)),
            pl.BlockSpec((1, 64), lambda b, r: (0, 0)),
        ],
        out_specs=[
            pl.BlockSpec((R, KNN), lambda b, r: (b * NB + r, 0)),
            pl.BlockSpec((R, 128), lambda b, r: (b * NB + r, 0)),
            pl.BlockSpec((R, 64), lambda b, r: (b * NB + r, 0)),
        ],
        out_shape=out_shape,
        scratch_shapes=[pltpu.VMEM((R, N), jnp.float32),
                        pltpu.VMEM((R, 128), jnp.float32)],
    )(x, wp, wq, bb)


CROWS = 8
NCH = None


def _sc_gather_max(idx, p, q):
    BN = idx.shape[0]
    info = plsc.get_sparse_core_info()
    nc, ns = info.num_cores, info.num_subcores
    nw = nc * ns
    rows_per_w = BN // nw
    nch = rows_per_w // CROWS
    mesh = plsc.VectorSubcoreMesh(core_axis_name="c", subcore_axis_name="s")

    @functools.partial(
        pl.kernel,
        mesh=mesh,
        out_type=jax.ShapeDtypeStruct((BN, 64), jnp.float32),
        scratch_types=[
            pltpu.VMEM((2, CROWS, KNN), jnp.int32),
            pltpu.VMEM((2, CROWS * KNN, 128), jnp.float32),
            pltpu.VMEM((2, CROWS, 64), jnp.float32),
            pltpu.VMEM((CROWS, 64), jnp.float32),
            pltpu.SemaphoreType.DMA,
            pltpu.SemaphoreType.DMA,
            pltpu.SemaphoreType.DMA,
            pltpu.SemaphoreType.DMA,
        ],
    )
    def body(idx_hbm, p_hbm, q_hbm, out_hbm, idx_v, rows_v, q_v, o_v,
             sg0, sg1, sq0, sq1):
        wid = lax.axis_index("s") * nc + lax.axis_index("c")
        row0 = wid * rows_per_w
        semg = (sg0, sg1)
        semq = (sq0, sq1)

        def stage(c, s):
            base = row0 + c * CROWS
            pltpu.sync_copy(idx_hbm.at[pl.ds(base, CROWS), :], idx_v.at[s])
            for j in range(CROWS):
                pltpu.async_copy(
                    p_hbm.at[idx_v.at[s, j]],
                    rows_v.at[s, pl.ds(j * KNN, KNN), :],
                    semg[s])
            pltpu.async_copy(q_hbm.at[pl.ds(base, CROWS), :], q_v.at[s],
                             semq[s])

        def drain(s):
            pltpu.make_async_copy(
                p_hbm.at[pl.ds(0, CROWS * KNN), :], rows_v.at[s],
                semg[s]).wait()
            pltpu.make_async_copy(
                q_hbm.at[pl.ds(0, CROWS), :], q_v.at[s], semq[s]).wait()

        def compute(c, s):
            base = row0 + c * CROWS
            for rr in range(CROWS):
                def jb(u, ms):
                    j0 = rr * KNN + u * 5
                    for uu in range(5):
                        ms = tuple(
                            jnp.maximum(ms[g],
                                        rows_v[s, j0 + uu, pl.ds(g * 16, 16)])
                            for g in range(4))
                    return ms
                init = tuple(jnp.full((16,), NEG, jnp.float32)
                             for _ in range(4))
                ms = lax.fori_loop(0, KNN // 5, jb, init)
                for g in range(4):
                    h = ms[g] + q_v[s, rr, pl.ds(g * 16, 16)]
                    o_v[rr, pl.ds(g * 16, 16)] = jnp.maximum(h, 0.2 * h)
            pltpu.sync_copy(o_v, out_hbm.at[pl.ds(base, CROWS), :])

        stage(0, 0)
        stage(1, 1)

        def outer(i, carry):
            c0 = 2 * i
            drain(0)
            compute(c0, 0)
            stage(c0 + 2, 0)
            drain(1)
            compute(c0 + 1, 1)
            stage(c0 + 3, 1)
            return carry

        lax.fori_loop(0, nch // 2 - 1, outer, 0)
        drain(0)
        compute(nch - 2, 0)
        drain(1)
        compute(nch - 1, 1)

    return body(idx, p, q)


def kernel(x, W, b):
    B, _, N = x.shape
    wp = jnp.zeros((3, 128), jnp.float32).at[:, :64].set(W[:, :3].T)
    wq = (W[:, 3:] - W[:, :3]).T
    bb = b.reshape(1, 64)
    ns = 4
    bs = B // ns
    outs = []
    for s in range(ns):
        xs = lax.slice_in_dim(x, s * bs, (s + 1) * bs, axis=0)
        idx, p, q = _knn_topk(xs, wp, wq, bb)
        outs.append(_sc_gather_max(idx, p, q))
    out_t = jnp.concatenate(outs, axis=0)
    return out_t.reshape(B, N, 64).transpose(0, 2, 1)


# speedup vs baseline: 1.1633x; 1.0599x over previous
import functools

import jax
import jax.numpy as jnp
from jax import lax
from jax.experimental import pallas as pl
from jax.experimental.pallas import tpu as pltpu
from jax.experimental.pallas import tpu_sc as plsc

KNN = 40
NEG = -3.0e38


def _knn_body(x_ref, wp_ref, wq_ref, b_ref, idx_ref, p_ref, q_ref, d_ref,
              c_ref):
    bidx = pl.program_id(0)
    r = pl.program_id(1)
    R = idx_ref.shape[0]
    N = x_ref.shape[2]
    xb = x_ref[0]
    xi = x_ref[0, :, pl.ds(r * R, R)]
    xx = jnp.sum(xb * xb, axis=0, keepdims=True)
    ones = jnp.ones((3, 1), jnp.float32)
    xxi = lax.dot_general(xi * xi, ones, (((0,), (0,)), ((), ())))
    cross = lax.dot_general(xi, xb, (((0,), (0,)), ((), ())))
    p_ref[...] = lax.dot_general(xi, wp_ref[...], (((0,), (0,)), ((), ())))
    q_ref[...] = lax.dot_general(xi, wq_ref[...], (((0,), (0,)), ((), ()))) \
        + b_ref[...]
    iota = lax.broadcasted_iota(jnp.int32, (R, N), 1)
    iota128 = lax.broadcasted_iota(jnp.int32, (R, 128), 1)
    base = bidx * N
    d_raw = 2.0 * cross - xxi - xx
    db = lax.bitcast_convert_type(d_raw, jnp.int32)
    pk = lax.bitcast_convert_type((db & -16) | (iota >> 7), jnp.float32)
    self_idx = r * R + lax.broadcasted_iota(jnp.int32, (R, 1), 0)
    d0 = jnp.where(iota == self_idx, NEG, pk)
    idx_ref[:, 0:1] = self_idx + base
    d_ref[...] = d0

    def colmax(a):
        c = a[:, 0:128]
        for v in range(1, 16):
            c = jnp.maximum(c, a[:, v * 128:(v + 1) * 128])
        return c

    c_ref[...] = colmax(d0)

    def peak(cm):
        vmax = jnp.max(cm, axis=1, keepdims=True)
        vstar = lax.bitcast_convert_type(vmax, jnp.int32) & 15
        lstar = jnp.min(jnp.where(cm == vmax, iota128, 128),
                        axis=1, keepdims=True)
        return vmax, vstar * 128 + lstar, lstar

    for t in range(1, KNN - 1, 2):
        cmax = c_ref[...]
        vmax1, am1, l1 = peak(cmax)
        idx_ref[:, t:t + 1] = am1 + base
        cmask = jnp.where(iota128 == l1, NEG, cmax)
        vmax2, am2, _ = peak(cmask)
        idx_ref[:, t + 1:t + 2] = am2 + base
        cur = d_ref[...]
        sel = (cur == vmax1) | (cur == vmax2)
        newd = jnp.where(sel, NEG, cur)
        d_ref[...] = newd
        c_ref[...] = colmax(newd)
    cmax = c_ref[...]
    _, am1, _ = peak(cmax)
    idx_ref[:, KNN - 1:KNN] = am1 + base


def _knn_topk(x, wp, wq, bb):
    B, _, N = x.shape
    R = 512
    NB = N // R
    out_shape = [
        jax.ShapeDtypeStruct((B * N, KNN), jnp.int32),
        jax.ShapeDtypeStruct((B * N, 128), jnp.float32),
        jax.ShapeDtypeStruct((B * N, 64), jnp.float32),
    ]
    return pl.pallas_call(
        _knn_body,
        grid=(B, NB),
        in_specs=[
            pl.BlockSpec((1, 3, N), lambda b, r: (b, 0, 0)),
            pl.BlockSpec((3, 128), lambda b, r: (0, 0)),
            pl.BlockSpec((3, 64), lambda b, r: (0, 0)),
            pl.BlockSpec((1, 64), lambda b, r: (0, 0)),
        ],
        out_specs=[
            pl.BlockSpec((R, KNN), lambda b, r: (b * NB + r, 0)),
            pl.BlockSpec((R, 128), lambda b, r: (b * NB + r, 0)),
            pl.BlockSpec((R, 64), lambda b, r: (b * NB + r, 0)),
        ],
        out_shape=out_shape,
        scratch_shapes=[pltpu.VMEM((R, N), jnp.float32),
                        pltpu.VMEM((R, 128), jnp.float32)],
    )(x, wp, wq, bb)


CROWS = 8
NCH = None


def _sc_gather_max(idx, p, q):
    BN = idx.shape[0]
    info = plsc.get_sparse_core_info()
    nc, ns = info.num_cores, info.num_subcores
    nw = nc * ns
    rows_per_w = BN // nw
    nch = rows_per_w // CROWS
    mesh = plsc.VectorSubcoreMesh(core_axis_name="c", subcore_axis_name="s")

    @functools.partial(
        pl.kernel,
        mesh=mesh,
        out_type=jax.ShapeDtypeStruct((BN, 64), jnp.float32),
        scratch_types=[
            pltpu.VMEM((2, CROWS, KNN), jnp.int32),
            pltpu.VMEM((2, CROWS * KNN, 128), jnp.float32),
            pltpu.VMEM((2, CROWS, 64), jnp.float32),
            pltpu.VMEM((CROWS, 64), jnp.float32),
            pltpu.SemaphoreType.DMA,
            pltpu.SemaphoreType.DMA,
            pltpu.SemaphoreType.DMA,
            pltpu.SemaphoreType.DMA,
        ],
    )
    def body(idx_hbm, p_hbm, q_hbm, out_hbm, idx_v, rows_v, q_v, o_v,
             sg0, sg1, sq0, sq1):
        wid = lax.axis_index("s") * nc + lax.axis_index("c")
        row0 = wid * rows_per_w
        semg = (sg0, sg1)
        semq = (sq0, sq1)

        def stage(c, s):
            base = row0 + c * CROWS
            pltpu.sync_copy(idx_hbm.at[pl.ds(base, CROWS), :], idx_v.at[s])
            for j in range(CROWS):
                pltpu.async_copy(
                    p_hbm.at[idx_v.at[s, j]],
                    rows_v.at[s, pl.ds(j * KNN, KNN), :],
                    semg[s])
            pltpu.async_copy(q_hbm.at[pl.ds(base, CROWS), :], q_v.at[s],
                             semq[s])

        def drain(s):
            pltpu.make_async_copy(
                p_hbm.at[pl.ds(0, CROWS * KNN), :], rows_v.at[s],
                semg[s]).wait()
            pltpu.make_async_copy(
                q_hbm.at[pl.ds(0, CROWS), :], q_v.at[s], semq[s]).wait()

        def compute(c, s):
            base = row0 + c * CROWS
            for rr in range(CROWS):
                def jb(u, ms):
                    j0 = rr * KNN + u * 5
                    for uu in range(5):
                        ms = tuple(
                            jnp.maximum(ms[g],
                                        rows_v[s, j0 + uu, pl.ds(g * 16, 16)])
                            for g in range(4))
                    return ms
                init = tuple(jnp.full((16,), NEG, jnp.float32)
                             for _ in range(4))
                ms = lax.fori_loop(0, KNN // 5, jb, init)
                for g in range(4):
                    h = ms[g] + q_v[s, rr, pl.ds(g * 16, 16)]
                    o_v[rr, pl.ds(g * 16, 16)] = jnp.maximum(h, 0.2 * h)
            pltpu.sync_copy(o_v, out_hbm.at[pl.ds(base, CROWS), :])

        stage(0, 0)
        stage(1, 1)

        def outer(i, carry):
            c0 = 2 * i
            drain(0)
            compute(c0, 0)
            stage(c0 + 2, 0)
            drain(1)
            compute(c0 + 1, 1)
            stage(c0 + 3, 1)
            return carry

        lax.fori_loop(0, nch // 2 - 1, outer, 0)
        drain(0)
        compute(nch - 2, 0)
        drain(1)
        compute(nch - 1, 1)

    return body(idx, p, q)


def kernel(x, W, b):
    B, _, N = x.shape
    wp = jnp.zeros((3, 128), jnp.float32).at[:, :64].set(W[:, :3].T)
    wq = (W[:, 3:] - W[:, :3]).T
    bb = b.reshape(1, 64)
    ns = 4
    bs = B // ns
    outs = []
    for s in range(ns):
        xs = lax.slice_in_dim(x, s * bs, (s + 1) * bs, axis=0)
        idx, p, q = _knn_topk(xs, wp, wq, bb)
        outs.append(_sc_gather_max(idx, p, q))
    out_t = jnp.concatenate(outs, axis=0)
    return out_t.reshape(B, N, 64).transpose(0, 2, 1)
